# Initial kernel scaffold; baseline (speedup 1.0000x reference)
import jax
import jax.numpy as jnp
from jax.experimental import pallas as pl


def kernel(token_ids, embedding_table):
    raise NotImplementedError("write your pallas kernel here")


# speedup vs baseline: 1.0229x; 1.0229x over previous
import functools

import jax
import jax.numpy as jnp
from jax import lax
from jax.experimental import pallas as pl
from jax.experimental.pallas import tpu as pltpu
from jax.experimental.pallas import tpu_sc as plsc

NUM_WORKERS = 32
CHUNK = 128


def _make_gather(B, D):
    assert B % (NUM_WORKERS * CHUNK) == 0
    per_w = B // NUM_WORKERS
    n_chunks = per_w // CHUNK
    mesh = plsc.VectorSubcoreMesh(core_axis_name="c", subcore_axis_name="s")

    @functools.partial(
        pl.kernel,
        mesh=mesh,
        compiler_params=pltpu.CompilerParams(use_tc_tiling_on_sc=False),
        out_type=jax.ShapeDtypeStruct((B, D), jnp.float32),
        scratch_types=[
            pltpu.VMEM((n_chunks, CHUNK), jnp.int32),
            pltpu.VMEM((CHUNK, D), jnp.float32),
            pltpu.SemaphoreType.DMA,
        ],
    )
    def gather_kernel(idx_hbm, table_hbm, out_hbm, idx_v, rows_v, gsem):
        wid = lax.axis_index("s") * 2 + lax.axis_index("c")
        base = wid * per_w
        pltpu.sync_copy(idx_hbm.at[wid], idx_v)

        def body(j, _):
            pltpu.async_copy(table_hbm.at[idx_v.at[j]], rows_v, gsem).wait()
            pltpu.sync_copy(rows_v, out_hbm.at[pl.ds(base + j * CHUNK, CHUNK)])
            return 0

        lax.fori_loop(0, n_chunks, body, 0)

    return gather_kernel


def kernel(token_ids, embedding_table):
    Bt, H = token_ids.shape
    V, D = embedding_table.shape
    B = Bt * H
    idx = token_ids.reshape(NUM_WORKERS, B // (NUM_WORKERS * CHUNK), CHUNK)
    idx = idx.astype(jnp.int32)
    out = _make_gather(B, D)(idx, embedding_table)
    return out.reshape(Bt, H, D)


# speedup vs baseline: 1.1112x; 1.0864x over previous
import functools

import jax
import jax.numpy as jnp
from jax import lax
from jax.experimental import pallas as pl
from jax.experimental.pallas import tpu as pltpu
from jax.experimental.pallas import tpu_sc as plsc

NUM_WORKERS = 32
CHUNK = 128
NSLOT = 8
INFLIGHT = 4


def _make_gather(B, D):
    assert B % (NUM_WORKERS * CHUNK) == 0
    per_w = B // NUM_WORKERS
    n_chunks = per_w // CHUNK
    assert n_chunks % NSLOT == 0
    mesh = plsc.VectorSubcoreMesh(core_axis_name="c", subcore_axis_name="s")

    @functools.partial(
        pl.kernel,
        mesh=mesh,
        compiler_params=pltpu.CompilerParams(use_tc_tiling_on_sc=False),
        out_type=jax.ShapeDtypeStruct((B, D), jnp.float32),
        scratch_types=[
            pltpu.VMEM((n_chunks, CHUNK), jnp.int32),
            pltpu.VMEM((NSLOT, CHUNK, D), jnp.float32),
            [pltpu.SemaphoreType.DMA] * NSLOT,
            [pltpu.SemaphoreType.DMA] * NSLOT,
        ],
    )
    def gather_kernel(idx_hbm, table_hbm, out_hbm, idx_v, rows_v, gsems, osems):
        wid = lax.axis_index("s") * 2 + lax.axis_index("c")
        base = wid * per_w
        pltpu.sync_copy(idx_hbm.at[wid], idx_v)

        def start_gather(g, b):
            pltpu.async_copy(table_hbm.at[idx_v.at[g]], rows_v.at[b], gsems[b])

        def wait_gather(g, b):
            pltpu.make_async_copy(table_hbm.at[idx_v.at[g]], rows_v.at[b],
                                  gsems[b]).wait()

        def start_out(g, b):
            pltpu.async_copy(rows_v.at[b],
                             out_hbm.at[pl.ds(base + g * CHUNK, CHUNK)],
                             osems[b])

        def wait_out(b):
            pltpu.make_async_copy(rows_v.at[b], out_hbm.at[pl.ds(0, CHUNK)],
                                  osems[b]).wait()

        for b in range(INFLIGHT):
            start_gather(b, b)

        def outer(t, _):
            g0 = t * NSLOT
            for b in range(NSLOT):
                g = g0 + b
                wait_gather(g, b)
                start_out(g, b)
                b2 = (b + INFLIGHT) % NSLOT
                g2 = g + INFLIGHT

                @pl.when(g2 < n_chunks)
                def _():
                    @pl.when(g2 >= NSLOT)
                    def _():
                        wait_out(b2)

                    start_gather(g2, b2)
            return 0

        lax.fori_loop(0, n_chunks // NSLOT, outer, 0)
        for b in range(NSLOT):
            wait_out(b)

    return gather_kernel


def kernel(token_ids, embedding_table):
    Bt, H = token_ids.shape
    V, D = embedding_table.shape
    B = Bt * H
    idx = token_ids.reshape(NUM_WORKERS, B // (NUM_WORKERS * CHUNK), CHUNK)
    idx = idx.astype(jnp.int32)
    out = _make_gather(B, D)(idx, embedding_table)
    return out.reshape(Bt, H, D)


# speedup vs baseline: 1.4104x; 1.2692x over previous
import functools

import jax
import jax.numpy as jnp
from jax import lax
from jax.experimental import pallas as pl
from jax.experimental.pallas import tpu as pltpu
from jax.experimental.pallas import tpu_sc as plsc

NW = 32
V = 1000000
D = 32
NB_FULL = 7808
SCR_ROWS = 250000


def _mesh():
    return plsc.VectorSubcoreMesh(core_axis_name="c", subcore_axis_name="s")


def _make_retile():
    @functools.partial(
        pl.kernel,
        mesh=_mesh(),
        compiler_params=pltpu.CompilerParams(needs_layout_passes=False),
        out_type=jax.ShapeDtypeStruct((SCR_ROWS, 128), jnp.float32),
        scratch_types=[
            pltpu.VMEM((32, 128), jnp.float32),
            pltpu.VMEM((32, 128), jnp.float32),
            pltpu.VMEM((32, 128), jnp.float32),
            pltpu.VMEM((32, 128), jnp.float32),
            pltpu.SemaphoreType.DMA,
            pltpu.SemaphoreType.DMA,
            pltpu.SemaphoreType.DMA,
            pltpu.SemaphoreType.DMA,
        ],
    )
    def retile_kernel(tab_hbm, tail_hbm, scr_hbm, nat_a, nat_b, tr_a, tr_b,
                      isem_a, isem_b, xsem_a, xsem_b):
        wid = lax.axis_index("s") * 2 + lax.axis_index("c")
        rows_lo = lax.iota(jnp.int32, 16)
        rows_hi = rows_lo + 16
        n_pairs = NB_FULL // NW // 2

        def start_in(i, nat, isem):
            pltpu.async_copy(tab_hbm.at[:, pl.ds(i * 128, 128)], nat, isem)

        def wait_in(nat, isem):
            pltpu.make_async_copy(tab_hbm.at[:, pl.ds(0, 128)], nat,
                                  isem).wait()

        def transpose_block(nat, tr, n_ids):
            def tbody(u, _):
                colv = rows_lo * 0 + u
                a = plsc.load_gather(nat, [rows_lo, colv])
                b = plsc.load_gather(nat, [rows_hi, colv])
                row = u // 4
                col = (u % 4) * 32
                tr[row, pl.ds(col, 16)] = a
                tr[row, pl.ds(col + 16, 16)] = b
                return 0

            lax.fori_loop(0, n_ids, tbody, 0)

        def start_out(i, tr, xsem):
            pltpu.async_copy(tr, scr_hbm.at[pl.ds(i * 32, 32), :], xsem)

        def wait_out(tr, xsem):
            pltpu.make_async_copy(tr, scr_hbm.at[pl.ds(0, 32), :],
                                  xsem).wait()

        start_in(wid, nat_a, isem_a)
        start_in(wid + NW, nat_b, isem_b)

        def outer(j2, _):
            i_a = wid + NW * (2 * j2)
            i_b = wid + NW * (2 * j2 + 1)

            wait_in(nat_a, isem_a)

            @pl.when(j2 >= 1)
            def _():
                wait_out(tr_a, xsem_a)

            transpose_block(nat_a, tr_a, 128)
            start_out(i_a, tr_a, xsem_a)

            @pl.when(j2 < n_pairs - 1)
            def _():
                start_in(i_a + 2 * NW, nat_a, isem_a)

            wait_in(nat_b, isem_b)

            @pl.when(j2 >= 1)
            def _():
                wait_out(tr_b, xsem_b)

            transpose_block(nat_b, tr_b, 128)
            start_out(i_b, tr_b, xsem_b)

            @pl.when(j2 < n_pairs - 1)
            def _():
                start_in(i_b + 2 * NW, nat_b, isem_b)

            return 0

        lax.fori_loop(0, n_pairs, outer, 0)
        wait_out(tr_a, xsem_a)
        wait_out(tr_b, xsem_b)

        @pl.when(wid < 4)
        def _():
            i = 7808 + wid
            pltpu.sync_copy(tab_hbm.at[:, pl.ds(i * 128, 128)], nat_a)
            transpose_block(nat_a, tr_a, 128)
            pltpu.sync_copy(tr_a, scr_hbm.at[pl.ds(i * 32, 32), :])

        @pl.when(wid == 4)
        def _():
            pltpu.sync_copy(tail_hbm, nat_a)
            transpose_block(nat_a, tr_a, 64)
            pltpu.sync_copy(tr_a.at[pl.ds(0, 16), :],
                            scr_hbm.at[pl.ds(249984, 16), :])

    return retile_kernel


def _make_gather(H, B):
    n_chunks = 50 * 4

    @functools.partial(
        pl.kernel,
        mesh=_mesh(),
        compiler_params=pltpu.CompilerParams(needs_layout_passes=False),
        out_type=jax.ShapeDtypeStruct((H, D, B), jnp.float32),
        scratch_types=[
            pltpu.VMEM((H, 512), jnp.int32),
            pltpu.VMEM((128, 128), jnp.float32),
            pltpu.VMEM((128, 128), jnp.float32),
            pltpu.VMEM((32, 128), jnp.float32),
            pltpu.VMEM((32, 128), jnp.float32),
            pltpu.VMEM((1, 128), jnp.int32),
            pltpu.VMEM((1, 128), jnp.int32),
            pltpu.VMEM((1, 128), jnp.int32),
            pltpu.VMEM((1, 128), jnp.int32),
            pltpu.SemaphoreType.DMA,
            pltpu.SemaphoreType.DMA,
            pltpu.SemaphoreType.DMA,
            pltpu.SemaphoreType.DMA,
        ],
    )
    def gather_kernel(ids_hbm, scr_hbm, out_hbm, ids_v, gbuf_a, gbuf_b,
                      tr_a, tr_b, idx_a, idx_b, off_a, off_b,
                      gsem_a, gsem_b, osem_a, osem_b):
        wid = lax.axis_index("s") * 2 + lax.axis_index("c")
        iota = lax.iota(jnp.int32, 16)
        pltpu.sync_copy(ids_hbm.at[:, pl.ds(512 * wid, 512)], ids_v)

        def prep(t, idx_ref, off_ref):
            h = lax.rem(t, 50)
            jj = t // 50
            for q in range(8):
                v = ids_v[h, pl.ds(128 * jj + 16 * q, 16)]
                idx_ref[0, pl.ds(16 * q, 16)] = lax.shift_right_logical(v, 2)
                off_ref[0, pl.ds(16 * q, 16)] = (v & 3) * 32

        def start_gather(idx_ref, gbuf, sem):
            pltpu.async_copy(scr_hbm.at[idx_ref.at[0]], gbuf, sem)

        def wait_gather(idx_ref, gbuf, sem):
            pltpu.make_async_copy(scr_hbm.at[idx_ref.at[0]], gbuf, sem).wait()

        def process(t, gbuf, off_ref, tr, osem):
            cols0 = tuple(off_ref[0, pl.ds(16 * q, 16)] for q in range(8))

            def fbody(f, cols):
                for q in range(8):
                    val = plsc.load_gather(gbuf, [iota + 16 * q, cols[q]])
                    tr[f, pl.ds(16 * q, 16)] = val
                return tuple(c + 1 for c in cols)

            lax.fori_loop(0, 32, fbody, cols0)
            h = lax.rem(t, 50)
            jj = t // 50
            b0 = 512 * wid + 128 * jj
            pltpu.async_copy(tr, out_hbm.at[h, :, pl.ds(b0, 128)], osem)

        def wait_out(tr, osem):
            pltpu.make_async_copy(tr, out_hbm.at[0, :, pl.ds(0, 128)],
                                  osem).wait()

        prep(0, idx_a, off_a)
        start_gather(idx_a, gbuf_a, gsem_a)
        prep(1, idx_b, off_b)
        start_gather(idx_b, gbuf_b, gsem_b)

        def outer(t2, _):
            t_a = 2 * t2
            t_b = t_a + 1

            wait_gather(idx_a, gbuf_a, gsem_a)

            @pl.when(t2 >= 1)
            def _():
                wait_out(tr_a, osem_a)

            process(t_a, gbuf_a, off_a, tr_a, osem_a)

            @pl.when(t2 < n_chunks // 2 - 1)
            def _():
                prep(t_a + 2, idx_a, off_a)
                start_gather(idx_a, gbuf_a, gsem_a)

            wait_gather(idx_b, gbuf_b, gsem_b)

            @pl.when(t2 >= 1)
            def _():
                wait_out(tr_b, osem_b)

            process(t_b, gbuf_b, off_b, tr_b, osem_b)

            @pl.when(t2 < n_chunks // 2 - 1)
            def _():
                prep(t_b + 2, idx_b, off_b)
                start_gather(idx_b, gbuf_b, gsem_b)

            return 0

        lax.fori_loop(0, n_chunks // 2, outer, 0)
        wait_out(tr_a, osem_a)
        wait_out(tr_b, osem_b)

    return gather_kernel


def kernel(token_ids, embedding_table):
    Bt, H = token_ids.shape
    ids_t = token_ids.T.astype(jnp.int32)
    tab_t = embedding_table.T
    tail = jnp.zeros((D, 128), jnp.float32).at[:, :64].set(
        tab_t[:, V - 64:])
    scr = _make_retile()(tab_t, tail)
    out = _make_gather(H, Bt)(ids_t, scr)
    return out.transpose(2, 0, 1)


# speedup vs baseline: 2.2359x; 1.5853x over previous
import functools

import jax
import jax.numpy as jnp
from jax import lax
from jax.experimental import pallas as pl
from jax.experimental.pallas import tpu as pltpu
from jax.experimental.pallas import tpu_sc as plsc

NW = 32
V = 1000000
D = 32
NB_FULL = 7808
SCR_ROWS = 250000


def _mesh():
    return plsc.VectorSubcoreMesh(core_axis_name="c", subcore_axis_name="s")


def _make_retile():
    @functools.partial(
        pl.kernel,
        mesh=_mesh(),
        compiler_params=pltpu.CompilerParams(needs_layout_passes=False),
        out_type=jax.ShapeDtypeStruct((SCR_ROWS, 128), jnp.float32),
        scratch_types=[
            pltpu.VMEM((32, 128), jnp.float32),
            pltpu.VMEM((32, 128), jnp.float32),
            pltpu.VMEM((32, 128), jnp.float32),
            pltpu.VMEM((32, 128), jnp.float32),
            pltpu.SemaphoreType.DMA,
            pltpu.SemaphoreType.DMA,
            pltpu.SemaphoreType.DMA,
            pltpu.SemaphoreType.DMA,
        ],
    )
    def retile_kernel(tab_hbm, tail_hbm, scr_hbm, nat_a, nat_b, tr_a, tr_b,
                      isem_a, isem_b, xsem_a, xsem_b):
        wid = lax.axis_index("s") * 2 + lax.axis_index("c")
        rows_lo = lax.iota(jnp.int32, 16)
        rows_hi = rows_lo + 16
        n_pairs = NB_FULL // NW // 2

        def start_in(i, nat, isem):
            pltpu.async_copy(tab_hbm.at[:, pl.ds(i * 128, 128)], nat, isem)

        def wait_in(nat, isem):
            pltpu.make_async_copy(tab_hbm.at[:, pl.ds(0, 128)], nat,
                                  isem).wait()

        def transpose_block(nat, tr, n_ids):
            @plsc.parallel_loop(0, n_ids, unroll=8)
            def _(u):
                colv = rows_lo * 0 + u
                a = plsc.load_gather(nat, [rows_lo, colv])
                b = plsc.load_gather(nat, [rows_hi, colv])
                row = u // 4
                col = (u % 4) * 32
                tr[row, pl.ds(col, 16)] = a
                tr[row, pl.ds(col + 16, 16)] = b

        def start_out(i, tr, xsem):
            pltpu.async_copy(tr, scr_hbm.at[pl.ds(i * 32, 32), :], xsem)

        def wait_out(tr, xsem):
            pltpu.make_async_copy(tr, scr_hbm.at[pl.ds(0, 32), :],
                                  xsem).wait()

        start_in(wid, nat_a, isem_a)
        start_in(wid + NW, nat_b, isem_b)

        def outer(j2, _):
            i_a = wid + NW * (2 * j2)
            i_b = wid + NW * (2 * j2 + 1)

            wait_in(nat_a, isem_a)

            @pl.when(j2 >= 1)
            def _():
                wait_out(tr_a, xsem_a)

            transpose_block(nat_a, tr_a, 128)
            start_out(i_a, tr_a, xsem_a)

            @pl.when(j2 < n_pairs - 1)
            def _():
                start_in(i_a + 2 * NW, nat_a, isem_a)

            wait_in(nat_b, isem_b)

            @pl.when(j2 >= 1)
            def _():
                wait_out(tr_b, xsem_b)

            transpose_block(nat_b, tr_b, 128)
            start_out(i_b, tr_b, xsem_b)

            @pl.when(j2 < n_pairs - 1)
            def _():
                start_in(i_b + 2 * NW, nat_b, isem_b)

            return 0

        lax.fori_loop(0, n_pairs, outer, 0)
        wait_out(tr_a, xsem_a)
        wait_out(tr_b, xsem_b)

        @pl.when(wid < 4)
        def _():
            i = 7808 + wid
            pltpu.sync_copy(tab_hbm.at[:, pl.ds(i * 128, 128)], nat_a)
            transpose_block(nat_a, tr_a, 128)
            pltpu.sync_copy(tr_a, scr_hbm.at[pl.ds(i * 32, 32), :])

        @pl.when(wid == 4)
        def _():
            pltpu.sync_copy(tail_hbm, nat_a)
            transpose_block(nat_a, tr_a, 64)
            pltpu.sync_copy(tr_a.at[pl.ds(0, 16), :],
                            scr_hbm.at[pl.ds(249984, 16), :])

    return retile_kernel


def _make_gather(H, B):
    n_chunks = 50 * 4

    @functools.partial(
        pl.kernel,
        mesh=_mesh(),
        compiler_params=pltpu.CompilerParams(needs_layout_passes=False),
        out_type=jax.ShapeDtypeStruct((H, D, B), jnp.float32),
        scratch_types=[
            pltpu.VMEM((H, 512), jnp.int32),
            pltpu.VMEM((128, 128), jnp.float32),
            pltpu.VMEM((128, 128), jnp.float32),
            pltpu.VMEM((32, 128), jnp.float32),
            pltpu.VMEM((32, 128), jnp.float32),
            pltpu.VMEM((1, 128), jnp.int32),
            pltpu.VMEM((1, 128), jnp.int32),
            pltpu.VMEM((1, 128), jnp.int32),
            pltpu.VMEM((1, 128), jnp.int32),
            pltpu.SemaphoreType.DMA,
            pltpu.SemaphoreType.DMA,
            pltpu.SemaphoreType.DMA,
            pltpu.SemaphoreType.DMA,
        ],
    )
    def gather_kernel(ids_hbm, scr_hbm, out_hbm, ids_v, gbuf_a, gbuf_b,
                      tr_a, tr_b, idx_a, idx_b, off_a, off_b,
                      gsem_a, gsem_b, osem_a, osem_b):
        wid = lax.axis_index("s") * 2 + lax.axis_index("c")
        iota = lax.iota(jnp.int32, 16)
        pltpu.sync_copy(ids_hbm.at[:, pl.ds(512 * wid, 512)], ids_v)

        def prep(t, idx_ref, off_ref):
            h = lax.rem(t, 50)
            jj = t // 50
            for q in range(8):
                v = ids_v[h, pl.ds(128 * jj + 16 * q, 16)]
                idx_ref[0, pl.ds(16 * q, 16)] = lax.shift_right_logical(v, 2)
                off_ref[0, pl.ds(16 * q, 16)] = (v & 3) * 32

        def start_gather(idx_ref, gbuf, sem):
            pltpu.async_copy(scr_hbm.at[idx_ref.at[0]], gbuf, sem)

        def wait_gather(idx_ref, gbuf, sem):
            pltpu.make_async_copy(scr_hbm.at[idx_ref.at[0]], gbuf, sem).wait()

        def process(t, gbuf, off_ref, tr, osem):
            rows_q = [iota + 16 * q for q in range(8)]
            offs_q = [off_ref[0, pl.ds(16 * q, 16)] for q in range(8)]

            @plsc.parallel_loop(0, 32, unroll=4)
            def _(f):
                for q in range(8):
                    val = plsc.load_gather(gbuf, [rows_q[q], offs_q[q] + f])
                    tr[f, pl.ds(16 * q, 16)] = val
            h = lax.rem(t, 50)
            jj = t // 50
            b0 = 512 * wid + 128 * jj
            pltpu.async_copy(tr, out_hbm.at[h, :, pl.ds(b0, 128)], osem)

        def wait_out(tr, osem):
            pltpu.make_async_copy(tr, out_hbm.at[0, :, pl.ds(0, 128)],
                                  osem).wait()

        prep(0, idx_a, off_a)
        start_gather(idx_a, gbuf_a, gsem_a)
        prep(1, idx_b, off_b)
        start_gather(idx_b, gbuf_b, gsem_b)

        def outer(t2, _):
            t_a = 2 * t2
            t_b = t_a + 1

            wait_gather(idx_a, gbuf_a, gsem_a)

            @pl.when(t2 >= 1)
            def _():
                wait_out(tr_a, osem_a)

            process(t_a, gbuf_a, off_a, tr_a, osem_a)

            @pl.when(t2 < n_chunks // 2 - 1)
            def _():
                prep(t_a + 2, idx_a, off_a)
                start_gather(idx_a, gbuf_a, gsem_a)

            wait_gather(idx_b, gbuf_b, gsem_b)

            @pl.when(t2 >= 1)
            def _():
                wait_out(tr_b, osem_b)

            process(t_b, gbuf_b, off_b, tr_b, osem_b)

            @pl.when(t2 < n_chunks // 2 - 1)
            def _():
                prep(t_b + 2, idx_b, off_b)
                start_gather(idx_b, gbuf_b, gsem_b)

            return 0

        lax.fori_loop(0, n_chunks // 2, outer, 0)
        wait_out(tr_a, osem_a)
        wait_out(tr_b, osem_b)

    return gather_kernel


def kernel(token_ids, embedding_table):
    Bt, H = token_ids.shape
    ids_t = token_ids.T.astype(jnp.int32)
    tab_t = embedding_table.T
    tail = jnp.zeros((D, 128), jnp.float32).at[:, :64].set(
        tab_t[:, V - 64:])
    scr = _make_retile()(tab_t, tail)
    out = _make_gather(H, Bt)(ids_t, scr)
    return out.transpose(2, 0, 1)
